# Initial kernel scaffold; baseline (speedup 1.0000x reference)
import jax
import jax.numpy as jnp
from jax.experimental import pallas as pl


def kernel(cos_cached, sin_cached, position_ids, batch_size, seq_len):
    raise NotImplementedError("write your pallas kernel here")


# speedup vs baseline: 5.1831x; 5.1831x over previous
import functools

import jax
import jax.numpy as jnp
from jax import lax
from jax.experimental import pallas as pl
from jax.experimental.pallas import tpu as pltpu
from jax.experimental.pallas import tpu_sc as plsc

HEAD_DIM = 128

_NUM_CORES = 2
_NUM_SUBCORES = 16
_NUM_WORKERS = _NUM_CORES * _NUM_SUBCORES
_CHUNK = 256


@functools.lru_cache(maxsize=None)
def _make_gather(n_rows):
    b_per_w = n_rows // _NUM_WORKERS
    n_chunks = b_per_w // _CHUNK
    mesh = plsc.VectorSubcoreMesh(core_axis_name="c", subcore_axis_name="s")

    @functools.partial(
        pl.kernel,
        mesh=mesh,
        out_type=[
            jax.ShapeDtypeStruct((n_rows, HEAD_DIM), jnp.float32),
            jax.ShapeDtypeStruct((n_rows, HEAD_DIM), jnp.float32),
        ],
        scratch_types=[
            pltpu.VMEM((b_per_w,), jnp.int32),
            pltpu.VMEM((_CHUNK, HEAD_DIM), jnp.float32),
            pltpu.VMEM((_CHUNK, HEAD_DIM), jnp.float32),
            pltpu.SemaphoreType.DMA,
        ],
    )
    def gather_kernel(cos_hbm, sin_hbm, idx_hbm, cos_out, sin_out,
                      idx_v, cbuf, sbuf, sem):
        wid = lax.axis_index("s") * _NUM_CORES + lax.axis_index("c")
        base = wid * b_per_w
        pltpu.sync_copy(idx_hbm.at[pl.ds(base, b_per_w)], idx_v)

        def body(i, carry):
            off = i * _CHUNK
            idx_slice = idx_v.at[pl.ds(off, _CHUNK)]
            c1 = pltpu.async_copy(cos_hbm.at[idx_slice], cbuf, sem)
            c2 = pltpu.async_copy(sin_hbm.at[idx_slice], sbuf, sem)
            c1.wait()
            c2.wait()
            pltpu.sync_copy(cbuf, cos_out.at[pl.ds(base + off, _CHUNK)])
            pltpu.sync_copy(sbuf, sin_out.at[pl.ds(base + off, _CHUNK)])
            return carry

        lax.fori_loop(0, n_chunks, body, 0)

    return gather_kernel


def kernel(cos_cached, sin_cached, position_ids, batch_size, seq_len):
    del batch_size, seq_len
    b, s = position_ids.shape
    cos_table = cos_cached[0, 0]
    sin_table = sin_cached[0, 0]
    idx = position_ids.reshape(-1)
    n_rows = b * s
    cos_flat, sin_flat = _make_gather(n_rows)(cos_table, sin_table, idx)
    cos = cos_flat.reshape(b, 1, s, HEAD_DIM)
    sin = sin_flat.reshape(b, 1, s, HEAD_DIM)
    return (cos, sin)


# speedup vs baseline: 5.3155x; 1.0255x over previous
import functools

import jax
import jax.numpy as jnp
from jax import lax
from jax.experimental import pallas as pl
from jax.experimental.pallas import tpu as pltpu
from jax.experimental.pallas import tpu_sc as plsc

HEAD_DIM = 128

_NUM_CORES = 2
_NUM_SUBCORES = 16
_NUM_WORKERS = _NUM_CORES * _NUM_SUBCORES
_CHUNK = 128
_NSLOT = 3


@functools.lru_cache(maxsize=None)
def _make_gather(n_rows):
    b_per_w = n_rows // _NUM_WORKERS
    n_chunks = b_per_w // _CHUNK
    mesh = plsc.VectorSubcoreMesh(core_axis_name="c", subcore_axis_name="s")

    buf_types = [pltpu.VMEM((_CHUNK, HEAD_DIM), jnp.float32)
                 for _ in range(2 * _NSLOT)]
    sem_types = [pltpu.SemaphoreType.DMA for _ in range(2 * _NSLOT)]

    @functools.partial(
        pl.kernel,
        mesh=mesh,
        out_type=[
            jax.ShapeDtypeStruct((n_rows, HEAD_DIM), jnp.float32),
            jax.ShapeDtypeStruct((n_rows, HEAD_DIM), jnp.float32),
        ],
        scratch_types=[pltpu.VMEM((b_per_w,), jnp.int32)]
                      + buf_types + sem_types,
    )
    def gather_kernel(cos_hbm, sin_hbm, idx_hbm, cos_out, sin_out,
                      idx_v, *bufs_and_sems):
        cbufs = bufs_and_sems[0:_NSLOT]
        sbufs = bufs_and_sems[_NSLOT:2 * _NSLOT]
        gsems = bufs_and_sems[2 * _NSLOT:3 * _NSLOT]
        wsems = bufs_and_sems[3 * _NSLOT:4 * _NSLOT]

        wid = lax.axis_index("s") * _NUM_CORES + lax.axis_index("c")
        base = wid * b_per_w
        pltpu.sync_copy(idx_hbm.at[pl.ds(base, b_per_w)], idx_v)

        def issue_gather(i):
            s = i % _NSLOT
            sl = idx_v.at[pl.ds(i * _CHUNK, _CHUNK)]
            return (pltpu.async_copy(cos_hbm.at[sl], cbufs[s], gsems[s]),
                    pltpu.async_copy(sin_hbm.at[sl], sbufs[s], gsems[s]))

        def issue_write(i):
            s = i % _NSLOT
            rows = pl.ds(base + i * _CHUNK, _CHUNK)
            return (pltpu.async_copy(cbufs[s], cos_out.at[rows], wsems[s]),
                    pltpu.async_copy(sbufs[s], sin_out.at[rows], wsems[s]))

        gh = {}
        wh = {}
        for i in range(min(2, n_chunks)):
            gh[i] = issue_gather(i)
        for i in range(n_chunks):
            if i >= 1:
                for h in wh.pop(i - 1):
                    h.wait()
            if i + 2 < n_chunks:
                gh[i + 2] = issue_gather(i + 2)
            for h in gh.pop(i):
                h.wait()
            wh[i] = issue_write(i)
        for h in wh.pop(n_chunks - 1):
            h.wait()

    return gather_kernel


def kernel(cos_cached, sin_cached, position_ids, batch_size, seq_len):
    del batch_size, seq_len
    b, s = position_ids.shape
    cos_table = cos_cached[0, 0]
    sin_table = sin_cached[0, 0]
    idx = position_ids.reshape(-1)
    n_rows = b * s
    cos_flat, sin_flat = _make_gather(n_rows)(cos_table, sin_table, idx)
    cos = cos_flat.reshape(b, 1, s, HEAD_DIM)
    sin = sin_flat.reshape(b, 1, s, HEAD_DIM)
    return (cos, sin)
